# Initial kernel scaffold; baseline (speedup 1.0000x reference)
import jax
import jax.numpy as jnp
from jax.experimental import pallas as pl


def kernel(x, W):
    raise NotImplementedError("write your pallas kernel here")


# speedup vs baseline: 1.3691x; 1.3691x over previous
import functools

import jax
import jax.numpy as jnp
from jax import lax
from jax.experimental import pallas as pl
from jax.experimental.pallas import tpu as pltpu
from jax.experimental.pallas import tpu_sc as plsc

BATCH = 1024
N = 4080
NV = N // 16
NW = 32
ROWS_PER_W = BATCH // NW
RC_BASE = 1


def _bf16r(f):
    u = plsc.bitcast(f, jnp.uint32)
    r = jnp.bitwise_and(
        u + jnp.uint32(0x7FFF) + jnp.bitwise_and(
            jnp.right_shift(u, jnp.uint32(16)), jnp.uint32(1)),
        jnp.uint32(0xFFFF0000))
    return plsc.bitcast(r, jnp.float32)


def _digits(k_i32, p):
    ku = plsc.bitcast(k_i32, jnp.uint32)
    d = jnp.bitwise_and(jnp.right_shift(ku, jnp.uint32(8 * p)), jnp.uint32(255))
    return d.astype(jnp.int32) + (256 * p)


def _make_run():
    mesh = plsc.VectorSubcoreMesh(core_axis_name="c", subcore_axis_name="s")

    @functools.partial(
        pl.kernel,
        mesh=mesh,
        out_type=jax.ShapeDtypeStruct((NW, 16), jnp.float32),
        compiler_params=pltpu.CompilerParams(needs_layout_passes=False),
        scratch_types=[
            pltpu.VMEM((N,), jnp.float32),
            pltpu.VMEM((N,), jnp.int32),
            pltpu.VMEM((N,), jnp.int32),
            pltpu.VMEM((N,), jnp.float32),
            pltpu.VMEM((4 * 256,), jnp.int32),
            pltpu.VMEM((4 * 256,), jnp.int32),
            pltpu.VMEM((16,), jnp.float32),
        ],
    )
    def run(x_hbm, w_hbm, out_hbm, xrow, bufA, bufB, wexp, hist, offs, ostage):
        wid = lax.axis_index("s") * 2 + lax.axis_index("c")

        pltpu.sync_copy(w_hbm, wexp)

        def maxbody(i, mv):
            return jnp.maximum(mv, wexp[pl.ds(i * 16, 16)])

        mv = lax.fori_loop(0, NV, maxbody,
                           jnp.full((16,), -jnp.inf, jnp.float32))
        wmax = jnp.max(mv)

        def expbody(i, se):
            e = jnp.exp(wexp[pl.ds(i * 16, 16)] - wmax)
            wexp[pl.ds(i * 16, 16)] = e
            return se + e

        sev = lax.fori_loop(0, NV, expbody, jnp.zeros((16,), jnp.float32))
        sumexp = jnp.sum(sev)

        def nbody(i, c):
            w = wexp[pl.ds(i * 16, 16)] / sumexp
            wexp[pl.ds(i * 16, 16)] = _bf16r(w)
            return c

        lax.fori_loop(0, NV, nbody, jnp.int32(0))

        def row_body(r, acc):
            row = wid * ROWS_PER_W + r
            pltpu.sync_copy(x_hbm.at[row], xrow)

            def zbody(i, c):
                hist[pl.ds(i * 16, 16)] = jnp.zeros((16,), jnp.int32)
                return c

            lax.fori_loop(0, 64, zbody, jnp.int32(0))

            def thbody(i, c):
                f = xrow[pl.ds(i * 16, 16)]
                b = plsc.bitcast(f, jnp.int32)
                m = jnp.right_shift(b, 31)
                k = jnp.bitwise_xor(
                    b, jnp.bitwise_and(jnp.bitwise_not(m),
                                       jnp.int32(0x7FFFFFFF)))
                bufA[pl.ds(i * 16, 16)] = k
                for p in range(4):
                    d = _digits(k, p)
                    rc, last = plsc.scan_count(d)
                    plsc.addupdate_scatter(hist, [d], rc + (1 - RC_BASE),
                                           mask=last)
                return c

            lax.fori_loop(0, NV, thbody, jnp.int32(0))

            for p in range(4):
                def sbody(i, carry, p=p):
                    h = hist[pl.ds(p * 256 + i * 16, 16)]
                    inc = plsc.cumsum(h)
                    offs[pl.ds(p * 256 + i * 16, 16)] = inc - h + carry
                    return carry + jnp.sum(h)

                lax.fori_loop(0, 16, sbody, jnp.int32(0))

            src, dst = bufA, bufB
            for p in range(4):
                def pbody(i, c, p=p, src=src, dst=dst):
                    k = src[pl.ds(i * 16, 16)]
                    d = _digits(k, p)
                    rc, last = plsc.scan_count(d)
                    base = plsc.load_gather(offs, [d])
                    plsc.store_scatter(dst, [base + (rc - RC_BASE)], k)
                    plsc.addupdate_scatter(offs, [d], rc + (1 - RC_BASE),
                                           mask=last)
                    return c

                lax.fori_loop(0, NV, pbody, jnp.int32(0))
                src, dst = dst, src

            def dbody(i, a):
                k = bufA[pl.ds(i * 16, 16)]
                mk = jnp.right_shift(k, 31)
                fb = jnp.bitwise_xor(
                    k, jnp.bitwise_and(jnp.bitwise_not(mk),
                                       jnp.int32(0x7FFFFFFF)))
                f = _bf16r(plsc.bitcast(fb, jnp.float32))
                return a + f * wexp[pl.ds(i * 16, 16)]

            return lax.fori_loop(0, NV, dbody, acc)

        acc = lax.fori_loop(0, ROWS_PER_W, row_body,
                            jnp.zeros((16,), jnp.float32))

        ostage[...] = acc * jnp.float32(1.0 / BATCH)
        pltpu.sync_copy(ostage, out_hbm.at[wid])

    return run


_run = None


def kernel(x, W):
    global _run
    if _run is None:
        _run = _make_run()
    partials = _run(x, W.reshape(N))
    return jnp.sum(partials)


# speedup vs baseline: 1.3930x; 1.0175x over previous
import functools

import jax
import jax.numpy as jnp
from jax import lax
from jax.experimental import pallas as pl
from jax.experimental.pallas import tpu as pltpu
from jax.experimental.pallas import tpu_sc as plsc

BATCH = 1024
N = 4080
NV = N // 16
NW = 32
ROWS_PER_W = BATCH // NW
RC_BASE = 1


def _bf16r(f):
    u = plsc.bitcast(f, jnp.uint32)
    r = jnp.bitwise_and(
        u + jnp.uint32(0x7FFF) + jnp.bitwise_and(
            jnp.right_shift(u, jnp.uint32(16)), jnp.uint32(1)),
        jnp.uint32(0xFFFF0000))
    return plsc.bitcast(r, jnp.float32)


def _digits(k_i32, p):
    ku = plsc.bitcast(k_i32, jnp.uint32)
    d = jnp.bitwise_and(jnp.right_shift(ku, jnp.uint32(8 * p)), jnp.uint32(255))
    return d.astype(jnp.int32) + (256 * p)


def _make_run():
    mesh = plsc.VectorSubcoreMesh(core_axis_name="c", subcore_axis_name="s")

    @functools.partial(
        pl.kernel,
        mesh=mesh,
        out_type=jax.ShapeDtypeStruct((NW, 16), jnp.float32),
        compiler_params=pltpu.CompilerParams(needs_layout_passes=False),
        scratch_types=[
            pltpu.VMEM((N,), jnp.float32),
            pltpu.VMEM((N,), jnp.int32),
            pltpu.VMEM((N,), jnp.int32),
            pltpu.VMEM((N,), jnp.float32),
            pltpu.VMEM((4 * 256,), jnp.int32),
            pltpu.VMEM((4 * 256,), jnp.int32),
            pltpu.VMEM((16,), jnp.float32),
        ],
    )
    def run(x_hbm, w_hbm, out_hbm, xrow, bufA, bufB, wexp, hist, offs, ostage):
        wid = lax.axis_index("s") * 2 + lax.axis_index("c")

        pltpu.sync_copy(w_hbm, wexp)

        def maxbody(i, mv):
            return jnp.maximum(mv, wexp[pl.ds(i * 16, 16)])

        mv = lax.fori_loop(0, NV, maxbody,
                           jnp.full((16,), -jnp.inf, jnp.float32), unroll=5)
        wmax = jnp.max(mv)

        def expbody(i, se):
            e = jnp.exp(wexp[pl.ds(i * 16, 16)] - wmax)
            wexp[pl.ds(i * 16, 16)] = e
            return se + e

        sev = lax.fori_loop(0, NV, expbody, jnp.zeros((16,), jnp.float32),
                            unroll=5)
        sumexp = jnp.sum(sev)

        def nbody(i, c):
            w = wexp[pl.ds(i * 16, 16)] / sumexp
            wexp[pl.ds(i * 16, 16)] = _bf16r(w)
            return c

        lax.fori_loop(0, NV, nbody, jnp.int32(0), unroll=5)

        def row_body(r, acc):
            row = wid * ROWS_PER_W + r
            pltpu.sync_copy(x_hbm.at[row], xrow)

            def zbody(i, c):
                hist[pl.ds(i * 16, 16)] = jnp.zeros((16,), jnp.int32)
                return c

            lax.fori_loop(0, 64, zbody, jnp.int32(0), unroll=8)

            def thbody(i, c):
                f = xrow[pl.ds(i * 16, 16)]
                b = plsc.bitcast(f, jnp.int32)
                m = jnp.right_shift(b, 31)
                k = jnp.bitwise_xor(
                    b, jnp.bitwise_and(jnp.bitwise_not(m),
                                       jnp.int32(0x7FFFFFFF)))
                bufA[pl.ds(i * 16, 16)] = k
                for p in range(4):
                    d = _digits(k, p)
                    rc, last = plsc.scan_count(d)
                    plsc.addupdate_scatter(hist, [d], rc + (1 - RC_BASE),
                                           mask=last)
                return c

            lax.fori_loop(0, NV, thbody, jnp.int32(0), unroll=5)

            for p in range(4):
                def sbody(i, carry, p=p):
                    h = hist[pl.ds(p * 256 + i * 16, 16)]
                    inc = plsc.cumsum(h)
                    offs[pl.ds(p * 256 + i * 16, 16)] = inc - h + carry
                    return carry + jnp.sum(h)

                lax.fori_loop(0, 16, sbody, jnp.int32(0), unroll=4)

            src, dst = bufA, bufB
            for p in range(4):
                def pbody(i, c, p=p, src=src, dst=dst):
                    k = src[pl.ds(i * 16, 16)]
                    d = _digits(k, p)
                    rc, last = plsc.scan_count(d)
                    base = plsc.load_gather(offs, [d])
                    plsc.store_scatter(dst, [base + (rc - RC_BASE)], k)
                    plsc.addupdate_scatter(offs, [d], rc + (1 - RC_BASE),
                                           mask=last)
                    return c

                lax.fori_loop(0, NV, pbody, jnp.int32(0), unroll=5)
                src, dst = dst, src

            def dbody(i, a):
                k = bufA[pl.ds(i * 16, 16)]
                mk = jnp.right_shift(k, 31)
                fb = jnp.bitwise_xor(
                    k, jnp.bitwise_and(jnp.bitwise_not(mk),
                                       jnp.int32(0x7FFFFFFF)))
                f = _bf16r(plsc.bitcast(fb, jnp.float32))
                return a + f * wexp[pl.ds(i * 16, 16)]

            return lax.fori_loop(0, NV, dbody, acc, unroll=5)

        acc = lax.fori_loop(0, ROWS_PER_W, row_body,
                            jnp.zeros((16,), jnp.float32))

        ostage[...] = acc * jnp.float32(1.0 / BATCH)
        pltpu.sync_copy(ostage, out_hbm.at[wid])

    return run


_run = None


def kernel(x, W):
    global _run
    if _run is None:
        _run = _make_run()
    partials = _run(x, W.reshape(N))
    return jnp.sum(partials)


# speedup vs baseline: 1.3966x; 1.0026x over previous
import functools

import jax
import jax.numpy as jnp
from jax import lax
from jax.experimental import pallas as pl
from jax.experimental.pallas import tpu as pltpu
from jax.experimental.pallas import tpu_sc as plsc

BATCH = 1024
N = 4080
NV = N // 16
NW = 32
ROWS_PER_W = BATCH // NW
RC_BASE = 1
II = 2
UN = 3


def _bf16r(f):
    u = plsc.bitcast(f, jnp.uint32)
    r = jnp.bitwise_and(
        u + jnp.uint32(0x7FFF) + jnp.bitwise_and(
            jnp.right_shift(u, jnp.uint32(16)), jnp.uint32(1)),
        jnp.uint32(0xFFFF0000))
    return plsc.bitcast(r, jnp.float32)


def _digits(k_i32, p):
    ku = plsc.bitcast(k_i32, jnp.uint32)
    d = jnp.bitwise_and(jnp.right_shift(ku, jnp.uint32(8 * p)), jnp.uint32(255))
    return d.astype(jnp.int32) + (256 * p)


def _key(f):
    b = plsc.bitcast(f, jnp.int32)
    m = jnp.right_shift(b, 31)
    return jnp.bitwise_xor(
        b, jnp.bitwise_and(jnp.bitwise_not(m), jnp.int32(0x7FFFFFFF)))


def _unkey(k):
    mk = jnp.right_shift(k, 31)
    fb = jnp.bitwise_xor(
        k, jnp.bitwise_and(jnp.bitwise_not(mk), jnp.int32(0x7FFFFFFF)))
    return plsc.bitcast(fb, jnp.float32)


def _make_run():
    mesh = plsc.VectorSubcoreMesh(core_axis_name="c", subcore_axis_name="s")

    per_slot = [
        pltpu.VMEM((N,), jnp.float32),
        pltpu.VMEM((N,), jnp.int32),
        pltpu.VMEM((N,), jnp.int32),
        pltpu.VMEM((4 * 256,), jnp.int32),
        pltpu.VMEM((4 * 256,), jnp.int32),
    ]

    @functools.partial(
        pl.kernel,
        mesh=mesh,
        out_type=jax.ShapeDtypeStruct((NW, 16), jnp.float32),
        compiler_params=pltpu.CompilerParams(needs_layout_passes=False),
        scratch_types=[
            pltpu.VMEM((N,), jnp.float32),
            pltpu.VMEM((16,), jnp.float32),
        ] + per_slot * II,
    )
    def run(x_hbm, w_hbm, out_hbm, wexp, ostage, *scr):
        slots = [scr[5 * j:5 * j + 5] for j in range(II)]
        wid = lax.axis_index("s") * 2 + lax.axis_index("c")

        pltpu.sync_copy(w_hbm, wexp)

        def maxbody(i, mv):
            return jnp.maximum(mv, wexp[pl.ds(i * 16, 16)])

        mv = lax.fori_loop(0, NV, maxbody,
                           jnp.full((16,), -jnp.inf, jnp.float32), unroll=5)
        wmax = jnp.max(mv)

        def expbody(i, se):
            e = jnp.exp(wexp[pl.ds(i * 16, 16)] - wmax)
            wexp[pl.ds(i * 16, 16)] = e
            return se + e

        sev = lax.fori_loop(0, NV, expbody, jnp.zeros((16,), jnp.float32),
                            unroll=5)
        sumexp = jnp.sum(sev)

        def nbody(i, c):
            w = wexp[pl.ds(i * 16, 16)] / sumexp
            wexp[pl.ds(i * 16, 16)] = _bf16r(w)
            return c

        lax.fori_loop(0, NV, nbody, jnp.int32(0), unroll=5)

        def group_body(g, acc):
            row0 = wid * ROWS_PER_W + g * II
            for j, (xrow, bufA, bufB, hist, offs) in enumerate(slots):
                pltpu.sync_copy(x_hbm.at[row0 + j], xrow)

            def zbody(i, c):
                for (xrow, bufA, bufB, hist, offs) in slots:
                    hist[pl.ds(i * 16, 16)] = jnp.zeros((16,), jnp.int32)
                return c

            lax.fori_loop(0, 64, zbody, jnp.int32(0), unroll=8)

            def thbody(i, c):
                for (xrow, bufA, bufB, hist, offs) in slots:
                    k = _key(xrow[pl.ds(i * 16, 16)])
                    bufA[pl.ds(i * 16, 16)] = k
                    for p in range(4):
                        d = _digits(k, p)
                        rc, last = plsc.scan_count(d)
                        plsc.addupdate_scatter(hist, [d], rc + (1 - RC_BASE),
                                               mask=last)
                return c

            lax.fori_loop(0, NV, thbody, jnp.int32(0), unroll=UN)

            for p in range(4):
                def sbody(i, carries, p=p):
                    out = []
                    for (xrow, bufA, bufB, hist, offs), carry in zip(
                            slots, carries):
                        h = hist[pl.ds(p * 256 + i * 16, 16)]
                        inc = plsc.cumsum(h)
                        offs[pl.ds(p * 256 + i * 16, 16)] = inc - h + carry
                        out.append(carry + jnp.sum(h))
                    return tuple(out)

                lax.fori_loop(0, 16, sbody, (jnp.int32(0),) * II, unroll=4)

            ab = [(s[1], s[2]) for s in slots]
            for p in range(4):
                def pbody(i, c, p=p, ab=tuple(ab)):
                    for (src, dst), (xrow, bufA, bufB, hist, offs) in zip(
                            ab, slots):
                        k = src[pl.ds(i * 16, 16)]
                        d = _digits(k, p)
                        rc, last = plsc.scan_count(d)
                        base = plsc.load_gather(offs, [d])
                        plsc.store_scatter(dst, [base + (rc - RC_BASE)], k)
                        plsc.addupdate_scatter(offs, [d], rc + (1 - RC_BASE),
                                               mask=last)
                    return c

                lax.fori_loop(0, NV, pbody, jnp.int32(0), unroll=UN)
                ab = [(b, a) for (a, b) in ab]

            def dbody(i, a):
                w = wexp[pl.ds(i * 16, 16)]
                for (xrow, bufA, bufB, hist, offs) in slots:
                    f = _bf16r(_unkey(bufA[pl.ds(i * 16, 16)]))
                    a = a + f * w
                return a

            return lax.fori_loop(0, NV, dbody, acc, unroll=UN)

        acc = lax.fori_loop(0, ROWS_PER_W // II, group_body,
                            jnp.zeros((16,), jnp.float32))

        ostage[...] = acc * jnp.float32(1.0 / BATCH)
        pltpu.sync_copy(ostage, out_hbm.at[wid])

    return run


_run = None


def kernel(x, W):
    global _run
    if _run is None:
        _run = _make_run()
    partials = _run(x, W.reshape(N))
    return jnp.sum(partials)


# speedup vs baseline: 2.7250x; 1.9512x over previous
import functools

import jax
import jax.numpy as jnp
from jax import lax
from jax.experimental import pallas as pl
from jax.experimental.pallas import tpu as pltpu
from jax.experimental.pallas import tpu_sc as plsc

BATCH = 1024
N = 4080
NV = N // 16
NW = 32
ROWS_PER_W = BATCH // NW
RC_BASE = 1
II = 2
NB = 5
NB_T = 3


def _bf16r(f):
    u = plsc.bitcast(f, jnp.uint32)
    r = jnp.bitwise_and(
        u + jnp.uint32(0x7FFF) + jnp.bitwise_and(
            jnp.right_shift(u, jnp.uint32(16)), jnp.uint32(1)),
        jnp.uint32(0xFFFF0000))
    return plsc.bitcast(r, jnp.float32)


def _digits(k_i32, p):
    ku = plsc.bitcast(k_i32, jnp.uint32)
    d = jnp.bitwise_and(jnp.right_shift(ku, jnp.uint32(8 * p)), jnp.uint32(255))
    return d.astype(jnp.int32) + (256 * p)


def _key(f):
    b = plsc.bitcast(f, jnp.int32)
    m = jnp.right_shift(b, 31)
    return jnp.bitwise_xor(
        b, jnp.bitwise_and(jnp.bitwise_not(m), jnp.int32(0x7FFFFFFF)))


def _unkey(k):
    mk = jnp.right_shift(k, 31)
    fb = jnp.bitwise_xor(
        k, jnp.bitwise_and(jnp.bitwise_not(mk), jnp.int32(0x7FFFFFFF)))
    return plsc.bitcast(fb, jnp.float32)


def _make_run():
    mesh = plsc.VectorSubcoreMesh(core_axis_name="c", subcore_axis_name="s")

    per_slot = [
        pltpu.VMEM((N,), jnp.float32),
        pltpu.VMEM((N,), jnp.int32),
        pltpu.VMEM((N,), jnp.int32),
        pltpu.VMEM((4 * 256,), jnp.int32),
        pltpu.VMEM((4 * 256,), jnp.int32),
    ]

    @functools.partial(
        pl.kernel,
        mesh=mesh,
        out_type=jax.ShapeDtypeStruct((NW, 16), jnp.float32),
        compiler_params=pltpu.CompilerParams(needs_layout_passes=False),
        scratch_types=[
            pltpu.VMEM((N,), jnp.float32),
            pltpu.VMEM((16,), jnp.float32),
        ] + per_slot * II,
    )
    def run(x_hbm, w_hbm, out_hbm, wexp, ostage, *scr):
        slots = [scr[5 * j:5 * j + 5] for j in range(II)]
        wid = lax.axis_index("s") * 2 + lax.axis_index("c")

        pltpu.sync_copy(w_hbm, wexp)

        def maxbody(i, mv):
            return jnp.maximum(mv, wexp[pl.ds(i * 16, 16)])

        mv = lax.fori_loop(0, NV, maxbody,
                           jnp.full((16,), -jnp.inf, jnp.float32), unroll=5)
        wmax = jnp.max(mv)

        def expbody(i, se):
            e = jnp.exp(wexp[pl.ds(i * 16, 16)] - wmax)
            wexp[pl.ds(i * 16, 16)] = e
            return se + e

        sev = lax.fori_loop(0, NV, expbody, jnp.zeros((16,), jnp.float32),
                            unroll=5)
        sumexp = jnp.sum(sev)

        def nbody(i, c):
            w = wexp[pl.ds(i * 16, 16)] / sumexp
            wexp[pl.ds(i * 16, 16)] = _bf16r(w)
            return c

        lax.fori_loop(0, NV, nbody, jnp.int32(0), unroll=5)

        def group_body(g, acc):
            row0 = wid * ROWS_PER_W + g * II
            for j, (xrow, bufA, bufB, hist, offs) in enumerate(slots):
                pltpu.sync_copy(x_hbm.at[row0 + j], xrow)

            def zbody(i, c):
                for (xrow, bufA, bufB, hist, offs) in slots:
                    hist[pl.ds(i * 16, 16)] = jnp.zeros((16,), jnp.int32)
                return c

            lax.fori_loop(0, 64, zbody, jnp.int32(0), unroll=8)

            def thbody(i, c):
                work = []
                for b in range(NB_T):
                    for (xrow, bufA, bufB, hist, offs) in slots:
                        k = _key(xrow[pl.ds((i * NB_T + b) * 16, 16)])
                        bufA[pl.ds((i * NB_T + b) * 16, 16)] = k
                        for p in range(4):
                            d = _digits(k, p)
                            rc, last = plsc.scan_count(d)
                            work.append((hist, d, rc, last))
                for (hist, d, rc, last) in work:
                    plsc.addupdate_scatter(hist, [d], rc + (1 - RC_BASE),
                                           mask=last)
                return c

            lax.fori_loop(0, NV // NB_T, thbody, jnp.int32(0))

            for p in range(4):
                for (xrow, bufA, bufB, hist, offs) in slots:
                    exs, tots = [], []
                    for i in range(16):
                        h = hist[pl.ds(p * 256 + i * 16, 16)]
                        inc = plsc.cumsum(h)
                        exs.append(inc - h)
                        tots.append(jnp.sum(h))
                    carry = jnp.int32(0)
                    for i in range(16):
                        offs[pl.ds(p * 256 + i * 16, 16)] = exs[i] + carry
                        carry = carry + tots[i]

            ab = [(s[1], s[2]) for s in slots]
            for p in range(4):
                def pbody(i, c, p=p, ab=tuple(ab)):
                    work = []
                    for b in range(NB):
                        for (src, dst), (xrow, bufA, bufB, hist, offs) in zip(
                                ab, slots):
                            k = src[pl.ds((i * NB + b) * 16, 16)]
                            d = _digits(k, p)
                            rc, last = plsc.scan_count(d)
                            work.append((dst, offs, k, d, rc, last))
                    for (dst, offs, k, d, rc, last) in work:
                        base = plsc.load_gather(offs, [d])
                        plsc.store_scatter(dst, [base + (rc - RC_BASE)], k)
                        plsc.addupdate_scatter(offs, [d], rc + (1 - RC_BASE),
                                               mask=last)
                    return c

                lax.fori_loop(0, NV // NB, pbody, jnp.int32(0))
                ab = [(b, a) for (a, b) in ab]

            def dbody(i, a):
                w = wexp[pl.ds(i * 16, 16)]
                for (xrow, bufA, bufB, hist, offs) in slots:
                    f = _bf16r(_unkey(bufA[pl.ds(i * 16, 16)]))
                    a = a + f * w
                return a

            return lax.fori_loop(0, NV, dbody, acc, unroll=5)

        acc = lax.fori_loop(0, ROWS_PER_W // II, group_body,
                            jnp.zeros((16,), jnp.float32))

        ostage[...] = acc * jnp.float32(1.0 / BATCH)
        pltpu.sync_copy(ostage, out_hbm.at[wid])

    return run


_run = None


def kernel(x, W):
    global _run
    if _run is None:
        _run = _make_run()
    partials = _run(x, W.reshape(N))
    return jnp.sum(partials)


# speedup vs baseline: 2.8403x; 1.0423x over previous
import functools

import jax
import jax.numpy as jnp
from jax import lax
from jax.experimental import pallas as pl
from jax.experimental.pallas import tpu as pltpu
from jax.experimental.pallas import tpu_sc as plsc

BATCH = 1024
N = 4080
NV = N // 16
NW = 32
ROWS_PER_W = BATCH // NW
RC_BASE = 1
II = 2
NB = 15
NB_T = 3


def _bf16r(f):
    u = plsc.bitcast(f, jnp.uint32)
    r = jnp.bitwise_and(
        u + jnp.uint32(0x7FFF) + jnp.bitwise_and(
            jnp.right_shift(u, jnp.uint32(16)), jnp.uint32(1)),
        jnp.uint32(0xFFFF0000))
    return plsc.bitcast(r, jnp.float32)


def _digits(k_i32, p):
    ku = plsc.bitcast(k_i32, jnp.uint32)
    d = jnp.bitwise_and(jnp.right_shift(ku, jnp.uint32(8 * p)), jnp.uint32(255))
    return d.astype(jnp.int32) + (256 * p)


def _key(f):
    b = plsc.bitcast(f, jnp.int32)
    m = jnp.right_shift(b, 31)
    return jnp.bitwise_xor(
        b, jnp.bitwise_and(jnp.bitwise_not(m), jnp.int32(0x7FFFFFFF)))


def _unkey(k):
    mk = jnp.right_shift(k, 31)
    fb = jnp.bitwise_xor(
        k, jnp.bitwise_and(jnp.bitwise_not(mk), jnp.int32(0x7FFFFFFF)))
    return plsc.bitcast(fb, jnp.float32)


def _make_run():
    mesh = plsc.VectorSubcoreMesh(core_axis_name="c", subcore_axis_name="s")

    per_slot = [
        pltpu.VMEM((N,), jnp.float32),
        pltpu.VMEM((N,), jnp.int32),
        pltpu.VMEM((N,), jnp.int32),
        pltpu.VMEM((4 * 256,), jnp.int32),
        pltpu.VMEM((4 * 256,), jnp.int32),
    ]

    @functools.partial(
        pl.kernel,
        mesh=mesh,
        out_type=jax.ShapeDtypeStruct((NW, 16), jnp.float32),
        compiler_params=pltpu.CompilerParams(needs_layout_passes=False),
        scratch_types=[
            pltpu.VMEM((N,), jnp.float32),
            pltpu.VMEM((16,), jnp.float32),
        ] + per_slot * II,
    )
    def run(x_hbm, w_hbm, out_hbm, wexp, ostage, *scr):
        slots = [scr[5 * j:5 * j + 5] for j in range(II)]
        wid = lax.axis_index("s") * 2 + lax.axis_index("c")

        pltpu.sync_copy(w_hbm, wexp)

        def maxbody(i, mv):
            return jnp.maximum(mv, wexp[pl.ds(i * 16, 16)])

        mv = lax.fori_loop(0, NV, maxbody,
                           jnp.full((16,), -jnp.inf, jnp.float32), unroll=5)
        wmax = jnp.max(mv)

        def expbody(i, se):
            e = jnp.exp(wexp[pl.ds(i * 16, 16)] - wmax)
            wexp[pl.ds(i * 16, 16)] = e
            return se + e

        sev = lax.fori_loop(0, NV, expbody, jnp.zeros((16,), jnp.float32),
                            unroll=5)
        sumexp = jnp.sum(sev)

        def nbody(i, c):
            w = wexp[pl.ds(i * 16, 16)] / sumexp
            wexp[pl.ds(i * 16, 16)] = _bf16r(w)
            return c

        lax.fori_loop(0, NV, nbody, jnp.int32(0), unroll=5)

        def group_body(g, acc):
            row0 = wid * ROWS_PER_W + g * II
            for j, (xrow, bufA, bufB, hist, offs) in enumerate(slots):
                pltpu.sync_copy(x_hbm.at[row0 + j], xrow)

            def zbody(i, c):
                for (xrow, bufA, bufB, hist, offs) in slots:
                    hist[pl.ds(i * 16, 16)] = jnp.zeros((16,), jnp.int32)
                return c

            lax.fori_loop(0, 64, zbody, jnp.int32(0), unroll=8)

            def thbody(i, c):
                work = []
                for b in range(NB_T):
                    for (xrow, bufA, bufB, hist, offs) in slots:
                        k = _key(xrow[pl.ds((i * NB_T + b) * 16, 16)])
                        bufA[pl.ds((i * NB_T + b) * 16, 16)] = k
                        for p in range(4):
                            d = _digits(k, p)
                            rc, last = plsc.scan_count(d)
                            work.append((hist, d, rc, last))
                for (hist, d, rc, last) in work:
                    plsc.addupdate_scatter(hist, [d], rc + (1 - RC_BASE),
                                           mask=last)
                return c

            lax.fori_loop(0, NV // NB_T, thbody, jnp.int32(0))

            for p in range(4):
                for (xrow, bufA, bufB, hist, offs) in slots:
                    exs, tots = [], []
                    for i in range(16):
                        h = hist[pl.ds(p * 256 + i * 16, 16)]
                        inc = plsc.cumsum(h)
                        exs.append(inc - h)
                        tots.append(jnp.sum(h))
                    carry = jnp.int32(0)
                    for i in range(16):
                        offs[pl.ds(p * 256 + i * 16, 16)] = exs[i] + carry
                        carry = carry + tots[i]

            ab = [(s[1], s[2]) for s in slots]
            for p in range(4):
                def pbody(i, c, p=p, ab=tuple(ab)):
                    work = []
                    for b in range(NB):
                        for (src, dst), (xrow, bufA, bufB, hist, offs) in zip(
                                ab, slots):
                            k = src[pl.ds((i * NB + b) * 16, 16)]
                            d = _digits(k, p)
                            rc, last = plsc.scan_count(d)
                            work.append((dst, offs, k, d, rc, last))
                    for (dst, offs, k, d, rc, last) in work:
                        base = plsc.load_gather(offs, [d])
                        plsc.store_scatter(dst, [base + (rc - RC_BASE)], k)
                        plsc.addupdate_scatter(offs, [d], rc + (1 - RC_BASE),
                                               mask=last)
                    return c

                lax.fori_loop(0, NV // NB, pbody, jnp.int32(0))
                ab = [(b, a) for (a, b) in ab]

            def dbody(i, a):
                w = wexp[pl.ds(i * 16, 16)]
                for (xrow, bufA, bufB, hist, offs) in slots:
                    f = _bf16r(_unkey(bufA[pl.ds(i * 16, 16)]))
                    a = a + f * w
                return a

            return lax.fori_loop(0, NV, dbody, acc, unroll=5)

        acc = lax.fori_loop(0, ROWS_PER_W // II, group_body,
                            jnp.zeros((16,), jnp.float32))

        ostage[...] = acc * jnp.float32(1.0 / BATCH)
        pltpu.sync_copy(ostage, out_hbm.at[wid])

    return run


_run = None


def kernel(x, W):
    global _run
    if _run is None:
        _run = _make_run()
    partials = _run(x, W.reshape(N))
    return jnp.sum(partials)


# speedup vs baseline: 2.8732x; 1.0116x over previous
import functools

import jax
import jax.numpy as jnp
from jax import lax
from jax.experimental import pallas as pl
from jax.experimental.pallas import tpu as pltpu
from jax.experimental.pallas import tpu_sc as plsc

BATCH = 1024
N = 4080
NV = N // 16
NW = 32
ROWS_PER_W = BATCH // NW
RC_BASE = 1
II = 2
NB = 15
NB_T = 5


def _bf16r(f):
    u = plsc.bitcast(f, jnp.uint32)
    r = jnp.bitwise_and(
        u + jnp.uint32(0x7FFF) + jnp.bitwise_and(
            jnp.right_shift(u, jnp.uint32(16)), jnp.uint32(1)),
        jnp.uint32(0xFFFF0000))
    return plsc.bitcast(r, jnp.float32)


def _digits(k_i32, p):
    ku = plsc.bitcast(k_i32, jnp.uint32)
    d = jnp.bitwise_and(jnp.right_shift(ku, jnp.uint32(8 * p)), jnp.uint32(255))
    return d.astype(jnp.int32) + (256 * p)


def _key(f):
    b = plsc.bitcast(f, jnp.int32)
    m = jnp.right_shift(b, 31)
    return jnp.bitwise_xor(
        b, jnp.bitwise_and(jnp.bitwise_not(m), jnp.int32(0x7FFFFFFF)))


def _unkey(k):
    mk = jnp.right_shift(k, 31)
    fb = jnp.bitwise_xor(
        k, jnp.bitwise_and(jnp.bitwise_not(mk), jnp.int32(0x7FFFFFFF)))
    return plsc.bitcast(fb, jnp.float32)


def _make_run():
    mesh = plsc.VectorSubcoreMesh(core_axis_name="c", subcore_axis_name="s")

    per_slot = [
        pltpu.VMEM((N,), jnp.float32),
        pltpu.VMEM((N,), jnp.int32),
        pltpu.VMEM((N,), jnp.int32),
        pltpu.VMEM((4 * 256,), jnp.int32),
        pltpu.VMEM((4 * 256,), jnp.int32),
    ]

    @functools.partial(
        pl.kernel,
        mesh=mesh,
        out_type=jax.ShapeDtypeStruct((NW, 16), jnp.float32),
        compiler_params=pltpu.CompilerParams(needs_layout_passes=False),
        scratch_types=[
            pltpu.VMEM((N,), jnp.float32),
            pltpu.VMEM((16,), jnp.float32),
        ] + per_slot * II,
    )
    def run(x_hbm, w_hbm, out_hbm, wexp, ostage, *scr):
        slots = [scr[5 * j:5 * j + 5] for j in range(II)]
        wid = lax.axis_index("s") * 2 + lax.axis_index("c")

        pltpu.sync_copy(w_hbm, wexp)

        def maxbody(i, mv):
            return jnp.maximum(mv, wexp[pl.ds(i * 16, 16)])

        mv = lax.fori_loop(0, NV, maxbody,
                           jnp.full((16,), -jnp.inf, jnp.float32), unroll=5)
        wmax = jnp.max(mv)

        def expbody(i, se):
            e = jnp.exp(wexp[pl.ds(i * 16, 16)] - wmax)
            wexp[pl.ds(i * 16, 16)] = e
            return se + e

        sev = lax.fori_loop(0, NV, expbody, jnp.zeros((16,), jnp.float32),
                            unroll=5)
        sumexp = jnp.sum(sev)

        def nbody(i, c):
            w = wexp[pl.ds(i * 16, 16)] / sumexp
            wexp[pl.ds(i * 16, 16)] = _bf16r(w)
            return c

        lax.fori_loop(0, NV, nbody, jnp.int32(0), unroll=5)

        def group_body(g, acc):
            row0 = wid * ROWS_PER_W + g * II
            for j, (xrow, bufA, bufB, hist, offs) in enumerate(slots):
                pltpu.sync_copy(x_hbm.at[row0 + j], xrow)

            def zbody(i, c):
                for (xrow, bufA, bufB, hist, offs) in slots:
                    hist[pl.ds(i * 16, 16)] = jnp.zeros((16,), jnp.int32)
                return c

            lax.fori_loop(0, 64, zbody, jnp.int32(0), unroll=8)

            def thbody(i, c):
                work = []
                for b in range(NB_T):
                    for (xrow, bufA, bufB, hist, offs) in slots:
                        k = _key(xrow[pl.ds((i * NB_T + b) * 16, 16)])
                        bufA[pl.ds((i * NB_T + b) * 16, 16)] = k
                        for p in range(4):
                            d = _digits(k, p)
                            rc, last = plsc.scan_count(d)
                            work.append((hist, d, rc, last))
                for (hist, d, rc, last) in work:
                    plsc.addupdate_scatter(hist, [d], rc + (1 - RC_BASE),
                                           mask=last)
                return c

            lax.fori_loop(0, NV // NB_T, thbody, jnp.int32(0))

            for p in range(4):
                for (xrow, bufA, bufB, hist, offs) in slots:
                    exs, tots = [], []
                    for i in range(16):
                        h = hist[pl.ds(p * 256 + i * 16, 16)]
                        inc = plsc.cumsum(h)
                        exs.append(inc - h)
                        tots.append(jnp.sum(h))
                    carry = jnp.int32(0)
                    for i in range(16):
                        offs[pl.ds(p * 256 + i * 16, 16)] = exs[i] + carry
                        carry = carry + tots[i]

            ab = [(s[1], s[2]) for s in slots]
            for p in range(4):
                def pbody(i, c, p=p, ab=tuple(ab)):
                    work = []
                    for b in range(NB):
                        for (src, dst), (xrow, bufA, bufB, hist, offs) in zip(
                                ab, slots):
                            k = src[pl.ds((i * NB + b) * 16, 16)]
                            d = _digits(k, p)
                            rc, last = plsc.scan_count(d)
                            work.append((dst, offs, k, d, rc, last))
                    for (dst, offs, k, d, rc, last) in work:
                        base = plsc.load_gather(offs, [d])
                        plsc.store_scatter(dst, [base + (rc - RC_BASE)], k)
                        plsc.addupdate_scatter(offs, [d], rc + (1 - RC_BASE),
                                               mask=last)
                    return c

                lax.fori_loop(0, NV // NB, pbody, jnp.int32(0))
                ab = [(b, a) for (a, b) in ab]

            def dbody(i, a):
                w = wexp[pl.ds(i * 16, 16)]
                for (xrow, bufA, bufB, hist, offs) in slots:
                    f = _bf16r(_unkey(bufA[pl.ds(i * 16, 16)]))
                    a = a + f * w
                return a

            return lax.fori_loop(0, NV, dbody, acc, unroll=5)

        acc = lax.fori_loop(0, ROWS_PER_W // II, group_body,
                            jnp.zeros((16,), jnp.float32))

        ostage[...] = acc * jnp.float32(1.0 / BATCH)
        pltpu.sync_copy(ostage, out_hbm.at[wid])

    return run


_run = None


def kernel(x, W):
    global _run
    if _run is None:
        _run = _make_run()
    partials = _run(x, W.reshape(N))
    return jnp.sum(partials)
